# baseline (speedup 1.0000x reference)
import functools

import jax
import jax.numpy as jnp
from jax import lax
from jax.experimental import pallas as pl
from jax.experimental.pallas import tpu as pltpu
from jax.experimental.pallas import tpu_sc as plsc


def _sc_geometry():
    try:
        info = plsc.get_sparse_core_info()
        return info.num_cores, info.num_subcores
    except Exception:
        return 2, 16


@functools.cache
def _build_gather(num_rows: int, dim: int, bsz: int, fields: int):
    NC, NS = _sc_geometry()
    NW = NC * NS
    BBLK = 128
    CSUB = 8
    assert bsz % (BBLK * NW) == 0 and dim % CSUB == 0
    blk_per_w = bsz // (BBLK * NW)
    assert blk_per_w == 1, "one 128-batch block per worker"
    n_bblk = bsz // BBLK
    NB = 2

    mesh = plsc.VectorSubcoreMesh(core_axis_name="c", subcore_axis_name="s")

    @functools.partial(
        pl.kernel,
        mesh=mesh,
        out_type=jax.ShapeDtypeStruct(
            (fields, dim // CSUB, n_bblk, CSUB, BBLK), jnp.float32),
        scratch_types=(
            [pltpu.VMEM((fields, BBLK), jnp.int32)]
            + [pltpu.VMEM((BBLK, dim), jnp.float32)] * NB
            + [pltpu.VMEM((dim // CSUB, 1, CSUB, BBLK), jnp.float32)] * NB
            + [pltpu.SemaphoreType.DMA] * (2 * NB)
        ),
        compiler_params=pltpu.CompilerParams(
            use_tc_tiling_on_sc=False, needs_layout_passes=False),
    )
    def gather_kernel(table_hbm, labt_hbm, out_hbm, *scratch):
        lab_v = scratch[0]
        rows_v = scratch[1:1 + NB]
        tr_v = scratch[1 + NB:1 + 2 * NB]
        row_sem = scratch[1 + 2 * NB:1 + 3 * NB]
        out_sem = scratch[1 + 3 * NB:1 + 4 * NB]
        wid = lax.axis_index("s") * NC + lax.axis_index("c")
        blk = wid * blk_per_w
        n_units = fields * blk_per_w

        pltpu.sync_copy(labt_hbm.at[:, pl.ds(blk * BBLK, BBLK)], lab_v)

        def gather_copy(f, s):
            return pltpu.make_async_copy(
                table_hbm.at[lab_v.at[f]], rows_v[s], row_sem[s])

        def out_copy(f, s):
            return pltpu.make_async_copy(
                tr_v[s],
                out_hbm.at[f, :, pl.ds(blk, 1)],
                out_sem[s])

        def transpose_unit(src, dst):
            for cb in range(dim // CSUB):
                for cl in range(CSUB):
                    c = cb * CSUB + cl
                    for k in range(BBLK // 16):
                        ridx = jnp.arange(16, dtype=jnp.int32) + 16 * k
                        cidx = jnp.full((16,), c, jnp.int32)
                        v = plsc.load_gather(src, [ridx, cidx])
                        dst[cb, 0, cl, pl.ds(16 * k, 16)] = v

        gather_copy(0, 0).start()

        def group(g, carry):
            for b_pos in range(NB):
                f = g * NB + b_pos
                s = b_pos
                sn = (b_pos + 1) % NB

                @pl.when(f + 1 < n_units)
                def _():
                    gather_copy(f + 1, sn).start()

                gather_copy(f, s).wait()

                @pl.when(f >= NB)
                def _():
                    out_copy(f - NB, s).wait()

                transpose_unit(rows_v[s], tr_v[s])
                out_copy(f, s).start()
            return carry

        lax.fori_loop(0, n_units // NB, group, 0)

        for j in range(n_units - NB, n_units):
            out_copy(j, j % NB).wait()

    return gather_kernel


def kernel(labels, embeddings):
    bsz, fields = labels.shape
    num_rows, dim = embeddings.shape
    fn = _build_gather(num_rows, dim, bsz, fields)
    out5 = fn(embeddings, labels.T)
    return out5.transpose(2, 4, 0, 1, 3).reshape(bsz, fields, dim)


# speedup vs baseline: 1.0016x; 1.0016x over previous
import functools

import jax
import jax.numpy as jnp
from jax import lax
from jax.experimental import pallas as pl
from jax.experimental.pallas import tpu as pltpu
from jax.experimental.pallas import tpu_sc as plsc


def _sc_geometry():
    try:
        info = plsc.get_sparse_core_info()
        return info.num_cores, info.num_subcores
    except Exception:
        return 2, 16


@functools.cache
def _build_gather(num_rows: int, dim: int, bsz: int, fields: int):
    NC, NS = _sc_geometry()
    NW = NC * NS
    BBLK = 128
    CSUB = 8
    assert bsz % (BBLK * NW) == 0 and dim % CSUB == 0
    blk_per_w = bsz // (BBLK * NW)
    assert blk_per_w == 1, "one 128-batch block per worker"
    n_bblk = bsz // BBLK
    NB = 2

    mesh = plsc.VectorSubcoreMesh(core_axis_name="c", subcore_axis_name="s")

    @functools.partial(
        pl.kernel,
        mesh=mesh,
        out_type=jax.ShapeDtypeStruct(
            (fields, dim // CSUB, n_bblk, CSUB, BBLK), jnp.float32),
        scratch_types=(
            [pltpu.VMEM((fields, BBLK), jnp.int32)]
            + [pltpu.VMEM((BBLK, dim), jnp.float32)] * NB
            + [pltpu.VMEM((dim // CSUB, 1, CSUB, BBLK), jnp.float32)] * NB
            + [pltpu.SemaphoreType.DMA] * (2 * NB)
        ),
        compiler_params=pltpu.CompilerParams(
            use_tc_tiling_on_sc=False, needs_layout_passes=False,
            disable_bounds_checks=True),
    )
    def gather_kernel(table_hbm, labt_hbm, out_hbm, *scratch):
        lab_v = scratch[0]
        rows_v = scratch[1:1 + NB]
        tr_v = scratch[1 + NB:1 + 2 * NB]
        row_sem = scratch[1 + 2 * NB:1 + 3 * NB]
        out_sem = scratch[1 + 3 * NB:1 + 4 * NB]
        wid = lax.axis_index("s") * NC + lax.axis_index("c")
        blk = wid * blk_per_w
        n_units = fields * blk_per_w

        pltpu.sync_copy(labt_hbm.at[:, pl.ds(blk * BBLK, BBLK)], lab_v)

        def gather_copy(f, s):
            return pltpu.make_async_copy(
                table_hbm.at[lab_v.at[f]], rows_v[s], row_sem[s])

        def out_copy(f, s):
            return pltpu.make_async_copy(
                tr_v[s],
                out_hbm.at[f, :, pl.ds(blk, 1)],
                out_sem[s])

        def transpose_unit(src, dst):
            for cb in range(dim // CSUB):
                for cl in range(CSUB):
                    c = cb * CSUB + cl
                    for k in range(BBLK // 16):
                        ridx = jnp.arange(16, dtype=jnp.int32) + 16 * k
                        cidx = jnp.full((16,), c, jnp.int32)
                        v = plsc.load_gather(src, [ridx, cidx])
                        dst[cb, 0, cl, pl.ds(16 * k, 16)] = v

        gather_copy(0, 0).start()

        def group(g, carry):
            for b_pos in range(NB):
                f = g * NB + b_pos
                s = b_pos
                sn = (b_pos + 1) % NB

                @pl.when(f + 1 < n_units)
                def _():
                    gather_copy(f + 1, sn).start()

                gather_copy(f, s).wait()

                @pl.when(f >= NB)
                def _():
                    out_copy(f - NB, s).wait()

                transpose_unit(rows_v[s], tr_v[s])
                out_copy(f, s).start()
            return carry

        lax.fori_loop(0, n_units // NB, group, 0)

        for j in range(n_units - NB, n_units):
            out_copy(j, j % NB).wait()

    return gather_kernel


def kernel(labels, embeddings):
    bsz, fields = labels.shape
    num_rows, dim = embeddings.shape
    fn = _build_gather(num_rows, dim, bsz, fields)
    out5 = fn(embeddings, labels.T)
    return out5.transpose(2, 4, 0, 1, 3).reshape(bsz, fields, dim)


# speedup vs baseline: 1.9585x; 1.9554x over previous
import functools

import jax
import jax.numpy as jnp
from jax import lax
from jax.experimental import pallas as pl
from jax.experimental.pallas import tpu as pltpu
from jax.experimental.pallas import tpu_sc as plsc


def _sc_geometry():
    try:
        info = plsc.get_sparse_core_info()
        return info.num_cores, info.num_subcores
    except Exception:
        return 2, 16


@functools.cache
def _build_gather(num_rows: int, dim: int, batch: int, fields: int):
    NC, NS = _sc_geometry()
    NW = NC * NS
    assert batch % NW == 0
    nb = batch // fields
    nb_per_w = nb // NW
    cb = 4
    while nb_per_w % cb != 0:
        cb //= 2
    chunk = cb * fields
    b_per_w = nb_per_w * fields
    n = nb_per_w // cb
    NB = 3
    assert n >= NB

    mesh = plsc.VectorSubcoreMesh(core_axis_name="c", subcore_axis_name="s")

    @functools.partial(
        pl.kernel,
        mesh=mesh,
        out_type=jax.ShapeDtypeStruct((batch // fields, fields, dim),
                                      jnp.float32),
        scratch_types=(
            [pltpu.VMEM((chunk,), jnp.int32)] * NB
            + [pltpu.VMEM((chunk, dim), jnp.float32)] * NB
            + [pltpu.SemaphoreType.DMA] * (3 * NB)
        ),
        compiler_params=pltpu.CompilerParams(use_tc_tiling_on_sc=False),
    )
    def gather_kernel(table_hbm, idx_hbm, out_hbm, *scratch):
        idx_v = scratch[0:NB]
        rows_v = scratch[NB:2 * NB]
        sems = scratch[2 * NB:]
        idx_sem = sems[0:NB]
        row_sem = sems[NB:2 * NB]
        out_sem = sems[2 * NB:3 * NB]
        wid = lax.axis_index("s") * NC + lax.axis_index("c")

        def idx_copy(i, b):
            return pltpu.make_async_copy(
                idx_hbm.at[pl.ds((wid * nb_per_w + i * cb) * fields, chunk)],
                idx_v[b], idx_sem[b])

        def gather_copy(b):
            return pltpu.make_async_copy(
                table_hbm.at[idx_v[b]], rows_v[b], row_sem[b])

        def out_copies(i, b):
            return [pltpu.make_async_copy(
                rows_v[b].at[pl.ds(j * fields, fields)],
                out_hbm.at[wid * nb_per_w + i * cb + j], out_sem[b])
                for j in range(cb)]

        def out_start(i, b):
            for c in out_copies(i, b):
                c.start()

        def out_wait(i, b):
            for c in out_copies(i, b):
                c.wait()

        for b in range(NB):
            idx_copy(b, b).start()

        n_groups = (n + 1 + NB - 1) // NB

        def group(g, carry):
            for b_pos in range(NB):
                i = g * NB + b_pos
                bj = (b_pos - 1) % NB

                @pl.when(i < n)
                def _():
                    @pl.when(i >= NB)
                    def _():
                        out_wait(i - NB, b_pos)

                    idx_copy(i, b_pos).wait()
                    gather_copy(b_pos).start()

                @pl.when(jnp.logical_and(i >= 1, i <= n))
                def _():
                    gather_copy(bj).wait()
                    out_start(i - 1, bj)

                    @pl.when(i - 1 + NB < n)
                    def _():
                        idx_copy(i - 1 + NB, bj).start()

            return carry

        lax.fori_loop(0, n_groups, group, 0)

        for j in range(n - NB, n):
            out_wait(j, j % NB)

    return gather_kernel


def kernel(labels, embeddings):
    bsz, fields = labels.shape
    num_rows, dim = embeddings.shape
    flat = labels.reshape(bsz * fields)
    fn = _build_gather(num_rows, dim, bsz * fields, fields)
    return fn(embeddings, flat)


# speedup vs baseline: 2.7825x; 1.4207x over previous
import functools

import jax
import jax.numpy as jnp
from jax import lax
from jax.experimental import pallas as pl
from jax.experimental.pallas import tpu as pltpu
from jax.experimental.pallas import tpu_sc as plsc


def _sc_geometry():
    try:
        info = plsc.get_sparse_core_info()
        return info.num_cores, info.num_subcores
    except Exception:
        return 2, 16


@functools.cache
def _build_gather(num_rows: int, dim: int, bsz: int, fields: int):
    NC, NS = _sc_geometry()
    NW = NC * NS
    BBLK = 128
    CSUB = 8
    assert bsz % (BBLK * NW) == 0 and dim % CSUB == 0
    blk_per_w = bsz // (BBLK * NW)
    assert blk_per_w == 1, "one 128-batch block per worker"
    n_bblk = bsz // BBLK
    NB = 2

    mesh = plsc.VectorSubcoreMesh(core_axis_name="c", subcore_axis_name="s")

    @functools.partial(
        pl.kernel,
        mesh=mesh,
        out_type=jax.ShapeDtypeStruct(
            (fields, dim // CSUB, n_bblk, CSUB, BBLK), jnp.float32),
        scratch_types=(
            [pltpu.VMEM((fields, BBLK), jnp.int32)]
            + [pltpu.VMEM((BBLK, dim), jnp.float32)] * NB
            + [pltpu.VMEM((dim // CSUB, 1, CSUB, BBLK + 1), jnp.float32)] * NB
            + [pltpu.SemaphoreType.DMA] * (2 * NB)
        ),
        compiler_params=pltpu.CompilerParams(
            use_tc_tiling_on_sc=False, needs_layout_passes=False,
            disable_bounds_checks=True),
    )
    def gather_kernel(table_hbm, labt_hbm, out_hbm, *scratch):
        lab_v = scratch[0]
        rows_v = scratch[1:1 + NB]
        tr_v = scratch[1 + NB:1 + 2 * NB]
        row_sem = scratch[1 + 2 * NB:1 + 3 * NB]
        out_sem = scratch[1 + 3 * NB:1 + 4 * NB]
        wid = lax.axis_index("s") * NC + lax.axis_index("c")
        blk = wid * blk_per_w
        n_units = fields * blk_per_w

        pltpu.sync_copy(labt_hbm.at[:, pl.ds(blk * BBLK, BBLK)], lab_v)

        def gather_copy(f, s):
            return pltpu.make_async_copy(
                table_hbm.at[lab_v.at[f]], rows_v[s], row_sem[s])

        def out_copy(f, s):
            return pltpu.make_async_copy(
                tr_v[s].at[:, :, :, pl.ds(0, BBLK)],
                out_hbm.at[f, :, pl.ds(blk, 1)],
                out_sem[s])

        lane = jnp.arange(16, dtype=jnp.int32)
        zero16 = jnp.zeros((16,), jnp.int32)

        def transpose_unit(src, dst):
            def bstep(b, carry):
                bv = jnp.full((16,), b, jnp.int32)
                for j in range(dim // 16):
                    c_hi = (lane + 16 * j) // CSUB
                    c_lo = (lane + 16 * j) % CSUB
                    v = src[b, pl.ds(16 * j, 16)]
                    plsc.store_scatter(dst, [c_hi, zero16, c_lo, bv], v)
                return carry

            lax.fori_loop(0, BBLK, bstep, 0)

        gather_copy(0, 0).start()

        def group(g, carry):
            for b_pos in range(NB):
                f = g * NB + b_pos
                s = b_pos
                sn = (b_pos + 1) % NB

                @pl.when(f + 1 < n_units)
                def _():
                    gather_copy(f + 1, sn).start()

                gather_copy(f, s).wait()

                @pl.when(f >= NB)
                def _():
                    out_copy(f - NB, s).wait()

                transpose_unit(rows_v[s], tr_v[s])
                out_copy(f, s).start()
            return carry

        lax.fori_loop(0, n_units // NB, group, 0)

        for j in range(n_units - NB, n_units):
            out_copy(j, j % NB).wait()

    return gather_kernel


def kernel(labels, embeddings):
    bsz, fields = labels.shape
    num_rows, dim = embeddings.shape
    fn = _build_gather(num_rows, dim, bsz, fields)
    out5 = fn(embeddings, labels.T)
    return out5.transpose(2, 4, 0, 1, 3).reshape(bsz, fields, dim)


# speedup vs baseline: 2.8935x; 1.0399x over previous
import functools

import jax
import jax.numpy as jnp
from jax import lax
from jax.experimental import pallas as pl
from jax.experimental.pallas import tpu as pltpu
from jax.experimental.pallas import tpu_sc as plsc


def _sc_geometry():
    try:
        info = plsc.get_sparse_core_info()
        return info.num_cores, info.num_subcores
    except Exception:
        return 2, 16


@functools.cache
def _build_gather(num_rows: int, dim: int, bsz: int, fields: int):
    NC, NS = _sc_geometry()
    NW = NC * NS
    BBLK = 128
    CSUB = 8
    assert bsz % (BBLK * NW) == 0 and dim % CSUB == 0
    blk_per_w = bsz // (BBLK * NW)
    assert blk_per_w == 1, "one 128-batch block per worker"
    n_bblk = bsz // BBLK
    NB = 2

    mesh = plsc.VectorSubcoreMesh(core_axis_name="c", subcore_axis_name="s")

    @functools.partial(
        pl.kernel,
        mesh=mesh,
        out_type=jax.ShapeDtypeStruct(
            (fields, dim // CSUB, n_bblk, CSUB, BBLK), jnp.float32),
        scratch_types=(
            [pltpu.VMEM((fields, BBLK), jnp.int32)]
            + [pltpu.VMEM((BBLK, dim), jnp.float32)] * NB
            + [pltpu.VMEM((dim // CSUB, 1, CSUB, BBLK + 1), jnp.float32)] * NB
            + [pltpu.SemaphoreType.DMA] * (2 * NB)
        ),
        compiler_params=pltpu.CompilerParams(
            use_tc_tiling_on_sc=False, needs_layout_passes=False,
            disable_bounds_checks=True),
    )
    def gather_kernel(table_hbm, labt_hbm, out_hbm, *scratch):
        lab_v = scratch[0]
        rows_v = scratch[1:1 + NB]
        tr_v = scratch[1 + NB:1 + 2 * NB]
        row_sem = scratch[1 + 2 * NB:1 + 3 * NB]
        out_sem = scratch[1 + 3 * NB:1 + 4 * NB]
        wid = lax.axis_index("s") * NC + lax.axis_index("c")
        blk = wid * blk_per_w
        n_units = fields * blk_per_w

        pltpu.sync_copy(labt_hbm.at[:, pl.ds(blk * BBLK, BBLK)], lab_v)

        def gather_copy(f, s):
            return pltpu.make_async_copy(
                table_hbm.at[lab_v.at[f]], rows_v[s], row_sem[s])

        def out_copy(f, s):
            return pltpu.make_async_copy(
                tr_v[s].at[:, :, :, pl.ds(0, BBLK)],
                out_hbm.at[f, :, pl.ds(blk, 1)],
                out_sem[s])

        lane = jnp.arange(16, dtype=jnp.int32)
        zero16 = jnp.zeros((16,), jnp.int32)

        def transpose_unit(src, dst):
            UNROLL = 4

            def bstep(bb, carry):
                for u in range(UNROLL):
                    b = bb * UNROLL + u
                    bv = jnp.full((16,), b, jnp.int32)
                    for j in range(dim // 16):
                        c_hi = (lane + 16 * j) // CSUB
                        c_lo = (lane + 16 * j) % CSUB
                        v = src[b, pl.ds(16 * j, 16)]
                        plsc.store_scatter(dst, [c_hi, zero16, c_lo, bv], v)
                return carry

            lax.fori_loop(0, BBLK // UNROLL, bstep, 0)

        gather_copy(0, 0).start()

        def group(g, carry):
            for b_pos in range(NB):
                f = g * NB + b_pos
                s = b_pos
                sn = (b_pos + 1) % NB

                @pl.when(f + 1 < n_units)
                def _():
                    gather_copy(f + 1, sn).start()

                gather_copy(f, s).wait()

                @pl.when(f >= NB)
                def _():
                    out_copy(f - NB, s).wait()

                transpose_unit(rows_v[s], tr_v[s])
                out_copy(f, s).start()
            return carry

        lax.fori_loop(0, n_units // NB, group, 0)

        for j in range(n_units - NB, n_units):
            out_copy(j, j % NB).wait()

    return gather_kernel


def kernel(labels, embeddings):
    bsz, fields = labels.shape
    num_rows, dim = embeddings.shape
    fn = _build_gather(num_rows, dim, bsz, fields)
    out5 = fn(embeddings, labels.T)
    return out5.transpose(2, 4, 0, 1, 3).reshape(bsz, fields, dim)


# speedup vs baseline: 4.5723x; 1.5802x over previous
import functools

import jax
import jax.numpy as jnp
from jax import lax
from jax.experimental import pallas as pl
from jax.experimental.pallas import tpu as pltpu
from jax.experimental.pallas import tpu_sc as plsc


def _sc_geometry():
    try:
        info = plsc.get_sparse_core_info()
        return info.num_cores, info.num_subcores
    except Exception:
        return 2, 16


@functools.cache
def _build_gather(num_rows: int, dim: int, bsz: int, fields: int):
    NC, NS = _sc_geometry()
    NW = NC * NS
    BBLK = 128
    CSUB = 8
    assert bsz % (BBLK * NW) == 0 and dim % CSUB == 0
    blk_per_w = bsz // (BBLK * NW)
    assert blk_per_w == 1, "one 128-batch block per worker"
    n_bblk = bsz // BBLK
    NB = 2

    mesh = plsc.VectorSubcoreMesh(core_axis_name="c", subcore_axis_name="s")

    @functools.partial(
        pl.kernel,
        mesh=mesh,
        out_type=jax.ShapeDtypeStruct(
            (fields, dim // CSUB, n_bblk, CSUB, BBLK), jnp.float32),
        scratch_types=(
            [pltpu.VMEM((fields, BBLK), jnp.int32)]
            + [pltpu.VMEM((BBLK, dim), jnp.float32)] * NB
            + [pltpu.VMEM((dim // CSUB, 1, CSUB, BBLK + 1), jnp.float32)] * NB
            + [pltpu.SemaphoreType.DMA] * (2 * NB)
        ),
        compiler_params=pltpu.CompilerParams(
            use_tc_tiling_on_sc=False, needs_layout_passes=False,
            disable_bounds_checks=True),
    )
    def gather_kernel(table_hbm, labt_hbm, out_hbm, *scratch):
        lab_v = scratch[0]
        rows_v = scratch[1:1 + NB]
        tr_v = scratch[1 + NB:1 + 2 * NB]
        row_sem = scratch[1 + 2 * NB:1 + 3 * NB]
        out_sem = scratch[1 + 3 * NB:1 + 4 * NB]
        wid = lax.axis_index("s") * NC + lax.axis_index("c")
        blk = wid * blk_per_w
        n_units = fields * blk_per_w

        pltpu.sync_copy(labt_hbm.at[:, pl.ds(blk * BBLK, BBLK)], lab_v)

        def gather_copy(f, s):
            return pltpu.make_async_copy(
                table_hbm.at[lab_v.at[f]], rows_v[s], row_sem[s])

        def out_copy(f, s):
            return pltpu.make_async_copy(
                tr_v[s].at[:, :, :, pl.ds(0, BBLK)],
                out_hbm.at[f, :, pl.ds(blk, 1)],
                out_sem[s])

        lane = jnp.arange(16, dtype=jnp.int32)
        zero16 = jnp.zeros((16,), jnp.int32)

        def transpose_unit(src, dst):
            @plsc.parallel_loop(0, BBLK, unroll=4)
            def bstep(b):
                bv = jnp.full((16,), b, jnp.int32)
                for j in range(dim // 16):
                    c_hi = (lane + 16 * j) // CSUB
                    c_lo = (lane + 16 * j) % CSUB
                    v = src[b, pl.ds(16 * j, 16)]
                    plsc.store_scatter(dst, [c_hi, zero16, c_lo, bv], v)

        gather_copy(0, 0).start()

        def group(g, carry):
            for b_pos in range(NB):
                f = g * NB + b_pos
                s = b_pos
                sn = (b_pos + 1) % NB

                @pl.when(f + 1 < n_units)
                def _():
                    gather_copy(f + 1, sn).start()

                gather_copy(f, s).wait()

                @pl.when(f >= NB)
                def _():
                    out_copy(f - NB, s).wait()

                transpose_unit(rows_v[s], tr_v[s])
                out_copy(f, s).start()
            return carry

        lax.fori_loop(0, n_units // NB, group, 0)

        for j in range(n_units - NB, n_units):
            out_copy(j, j % NB).wait()

    return gather_kernel


def kernel(labels, embeddings):
    bsz, fields = labels.shape
    num_rows, dim = embeddings.shape
    fn = _build_gather(num_rows, dim, bsz, fields)
    out5 = fn(embeddings, labels.T)
    return out5.transpose(2, 4, 0, 1, 3).reshape(bsz, fields, dim)
